# Initial kernel scaffold; baseline (speedup 1.0000x reference)
import jax
import jax.numpy as jnp
from jax import lax
from jax.experimental import pallas as pl
from jax.experimental.pallas import tpu as pltpu
from jax.experimental.pallas import tpu_sc as plsc
import functools

RES_F = 100000.0
EMBED = 64
NW = 32
B_PER_W = 1024
N_CHUNK = 8
CHUNK = 128


@functools.partial(
    pl.kernel,
    mesh=plsc.VectorSubcoreMesh(core_axis_name="c", subcore_axis_name="s"),
    out_type=jax.ShapeDtypeStruct((NW * B_PER_W, EMBED), jnp.float32),
    scratch_types=[
        pltpu.VMEM((B_PER_W,), jnp.float32),
        pltpu.VMEM((N_CHUNK, CHUNK), jnp.int32),
        pltpu.VMEM((B_PER_W, EMBED), jnp.float32),
        pltpu.SemaphoreType.DMA,
    ],
)
def _sc_embed(x_hbm, table_hbm, out_hbm, xv, idxv, rows, sem):
    wid = lax.axis_index("s") * 2 + lax.axis_index("c")
    base = wid * B_PER_W

    pltpu.sync_copy(x_hbm.at[pl.ds(base, B_PER_W)], xv)

    for j in range(N_CHUNK):
        def body(i, _):
            off = j * CHUNK + i * 16
            v = xv[pl.ds(off, 16)] * RES_F
            idxv[j, pl.ds(i * 16, 16)] = v.astype(jnp.int32)
            return 0
        lax.fori_loop(0, CHUNK // 16, body, 0)

    copies = []
    for j in range(N_CHUNK):
        copies.append(
            pltpu.async_copy(
                table_hbm.at[idxv.at[j]],
                rows.at[pl.ds(j * CHUNK, CHUNK)],
                sem,
            )
        )
    for c in copies:
        c.wait()

    pltpu.sync_copy(rows, out_hbm.at[pl.ds(base, B_PER_W)])


def kernel(x, table):
    out = _sc_embed(x.reshape(-1), table)
    return out.reshape(-1, 2 * EMBED)


# speedup vs baseline: 1.0250x; 1.0250x over previous
import jax
import jax.numpy as jnp
from jax import lax
from jax.experimental import pallas as pl
from jax.experimental.pallas import tpu as pltpu
from jax.experimental.pallas import tpu_sc as plsc
import functools

RES_F = 100000.0
EMBED = 64
NW = 32
B_PER_W = 1024
N_CHUNK = 8
CHUNK = 128


@functools.partial(
    pl.kernel,
    mesh=plsc.VectorSubcoreMesh(core_axis_name="c", subcore_axis_name="s"),
    out_type=jax.ShapeDtypeStruct((NW * B_PER_W, EMBED), jnp.float32),
    scratch_types=[
        pltpu.VMEM((B_PER_W,), jnp.float32),
        pltpu.VMEM((N_CHUNK, CHUNK), jnp.int32),
        pltpu.VMEM((B_PER_W, EMBED), jnp.float32),
        pltpu.SemaphoreType.DMA,
    ],
    compiler_params=pltpu.CompilerParams(use_tc_tiling_on_sc=False),
)
def _sc_embed(x_hbm, table_hbm, out_hbm, xv, idxv, rows, sem):
    wid = lax.axis_index("s") * 2 + lax.axis_index("c")
    base = wid * B_PER_W

    pltpu.sync_copy(x_hbm.at[pl.ds(base, B_PER_W)], xv)

    for j in range(N_CHUNK):
        def body(i, _):
            off = j * CHUNK + i * 16
            v = xv[pl.ds(off, 16)] * RES_F
            idxv[j, pl.ds(i * 16, 16)] = v.astype(jnp.int32)
            return 0
        lax.fori_loop(0, CHUNK // 16, body, 0)

    copies = []
    for j in range(N_CHUNK):
        copies.append(
            pltpu.async_copy(
                table_hbm.at[idxv.at[j]],
                rows.at[pl.ds(j * CHUNK, CHUNK)],
                sem,
            )
        )
    for c in copies:
        c.wait()

    pltpu.sync_copy(rows, out_hbm.at[pl.ds(base, B_PER_W)])


def kernel(x, table):
    out = _sc_embed(x.reshape(-1), table)
    return out.reshape(-1, 2 * EMBED)
